# baseline (speedup 1.0000x reference)
import functools

import jax
import jax.numpy as jnp
from jax import lax
from jax.experimental import pallas as pl
from jax.experimental.pallas import tpu as pltpu
from jax.experimental.pallas import tpu_sc as plsc

H = 128
K = 16
DA = 256
NEG = -3.0e38


def _prep_kernel(feats_ref, key_ref, sel_ref, wi_ref, bi_ref, wa_ref, wf_ref,
                 ba_ref, h_ref, sq_ref, a_ref, hk_ref):
    f = feats_ref[...]
    h = jnp.maximum(f @ wi_ref[...] + bi_ref[...], 0.0)
    kv = key_ref[...]
    h_ref[...] = h
    sq_ref[...] = jnp.sum(h * h, axis=1, keepdims=True)
    a_ref[...] = h @ wa_ref[...] + kv * wf_ref[...] + ba_ref[...]
    hk_ref[:, 0:H] = h
    hk_ref[:, H:DA] = kv * sel_ref[...]


def _knn_kernel(nreal, br, npad, hb_ref, ha_ref, sqb_ref, sqt_ref, idx_ref,
                k_ref, c_ref):
    i = pl.program_id(0)
    maxi = jnp.int32(0x7FFFFFFF)
    ngrp = npad // 640
    hb = hb_ref[...]
    d = sqb_ref[...] + sqt_ref[...] + lax.dot_general(
        -2.0 * hb, ha_ref[...], (((1,), (1,)), ((), ())),
        preferred_element_type=jnp.float32)
    rows = i * br + lax.broadcasted_iota(jnp.int32, (br, npad), 0)
    cols = lax.broadcasted_iota(jnp.int32, (br, npad), 1)
    rm = jnp.max(d, axis=1, keepdims=True)
    q = (jnp.maximum(d, 0.0) * (131071.0 / jnp.maximum(rm, 1e-20))
         ).astype(jnp.int32)
    key = (q << 14) | cols
    key = jnp.where((cols == rows) | (cols >= nreal), maxi, key)
    k_ref[...] = key
    sl = [key[:, j * 640:(j + 1) * 640] for j in range(ngrp)]
    m1 = sl[0]
    for j in range(1, ngrp):
        m1 = jnp.minimum(m1, sl[j])
    m2 = maxi
    for j in range(ngrp):
        m2 = jnp.minimum(m2, jnp.where(sl[j] == m1, maxi, sl[j]))
    m3 = maxi
    for j in range(ngrp):
        m3 = jnp.minimum(
            m3, jnp.where((sl[j] == m1) | (sl[j] == m2), maxi, sl[j]))
    c_ref[:, 0:640] = m1
    c_ref[:, 640:1280] = m2
    c_ref[:, 1280:1920] = m3
    prev = jnp.full((br, 1), -1, jnp.int32)
    for t in range(K):
        kv = c_ref[...]
        m = jnp.min(jnp.where(kv > prev, kv, maxi), axis=1, keepdims=True)
        idx_ref[:, t:t + 1] = m & 0x3FFF
        prev = m
    cnt = jnp.zeros((br, 640), jnp.int32)
    for j in range(ngrp):
        cnt = cnt + jnp.where(sl[j] <= prev, 1, 0)
    bad = jnp.any(jnp.sum(cnt, axis=1, keepdims=True) != K)

    @pl.when(bad)
    def _():
        prev = jnp.full((br, 1), -1, jnp.int32)
        for t in range(K):
            kv = k_ref[...]
            m = jnp.min(jnp.where(kv > prev, kv, maxi), axis=1, keepdims=True)
            idx_ref[:, t:t + 1] = m & 0x3FFF
            prev = m


def _edge1_kernel(bn, g_ref, a_ref, h_ref, key_ref, sel_ref, wb_ref, wf_ref,
                  wm2_ref, bm2_ref, wa2_ref, wf2_ref, ba2_ref, x1_ref, a2_ref,
                  hk2_ref):
    g = g_ref[...]
    a = a_ref[...]
    a3 = jnp.broadcast_to(a[:, None, :], (bn, K, 2 * H)).reshape(bn * K, 2 * H)
    pre = g[:, 0:H] @ wb_ref[...] + a3 - g[:, H:H + 1] * wf_ref[...]
    tmsg = jnp.maximum(pre[:, 0:H], 0.0) @ wm2_ref[...] + bm2_ref[...]
    gate = 1.0 / (1.0 + jnp.exp(-pre[:, H:2 * H]))
    msg = gate * tmsg + (1.0 - gate) * g[:, 0:H]
    s = jnp.sum(msg.reshape(bn, K, H), axis=1) * (1.0 / K)
    x = s + h_ref[...]
    x = jnp.where(x > 0, x, 0.2 * x)
    kv = key_ref[...]
    x1_ref[...] = x
    a2_ref[...] = x @ wa2_ref[...] + kv * wf2_ref[...] + ba2_ref[...]
    hk2_ref[:, 0:H] = x
    hk2_ref[:, H:DA] = kv * sel_ref[...]


def _edge2_kernel(nreal, bn, g_ref, a_ref, x1_ref, wb_ref, wf_ref, wm2_ref,
                  bm2_ref, pool_ref):
    i = pl.program_id(0)
    g = g_ref[...]
    a = a_ref[...]
    a3 = jnp.broadcast_to(a[:, None, :], (bn, K, 2 * H)).reshape(bn * K, 2 * H)
    pre = g[:, 0:H] @ wb_ref[...] + a3 - g[:, H:H + 1] * wf_ref[...]
    tmsg = jnp.maximum(pre[:, 0:H], 0.0) @ wm2_ref[...] + bm2_ref[...]
    gate = 1.0 / (1.0 + jnp.exp(-pre[:, H:2 * H]))
    msg = gate * tmsg + (1.0 - gate) * g[:, 0:H]
    s = jnp.sum(msg.reshape(bn, K, H), axis=1) * (1.0 / K)
    x1 = x1_ref[...]
    x2 = s + x1
    x2 = jnp.where(x2 > 0, x2, 0.2 * x2)
    xc = jnp.concatenate([x1, x2], axis=1)
    rows = i * bn + lax.broadcasted_iota(jnp.int32, (bn, 1), 0)
    xm = jnp.where(rows < nreal, xc, NEG)
    bmax = jnp.max(xm, axis=0, keepdims=True)

    @pl.when(i == 0)
    def _():
        pool_ref[...] = bmax

    @pl.when(i > 0)
    def _():
        pool_ref[...] = jnp.maximum(pool_ref[...], bmax)


def _cls_kernel(p_ref, w1_ref, b1_ref, w2_ref, b2_ref, o_ref):
    z = jnp.maximum(p_ref[...] @ w1_ref[...] + b1_ref[...], 0.0)
    z = z @ w2_ref[...] + b2_ref[...]
    m = jnp.max(z, axis=1, keepdims=True)
    e = jnp.exp(z - m)
    o_ref[...] = (z - m) - jnp.log(jnp.sum(e, axis=1, keepdims=True))


def _sc_gather(table, idx):
    ne = idx.shape[0]
    d = table.shape[1]
    nw = 32
    bpw = ne // nw
    ch = 160
    nch = bpw // ch
    mesh = plsc.VectorSubcoreMesh(core_axis_name="c", subcore_axis_name="s")

    @functools.partial(
        pl.kernel,
        out_type=jax.ShapeDtypeStruct((ne, d), jnp.float32),
        mesh=mesh,
        scratch_types=[
            pltpu.VMEM((bpw,), jnp.int32),
            pltpu.VMEM((2, ch, d), jnp.float32),
            pltpu.SemaphoreType.DMA,
            pltpu.SemaphoreType.DMA,
            pltpu.SemaphoreType.DMA,
        ],
    )
    def k(table_hbm, idx_hbm, out_hbm, idx_v, rows_v, gsem, wsem0, wsem1):
        wid = lax.axis_index("s") * 2 + lax.axis_index("c")
        base = wid * bpw
        pltpu.sync_copy(idx_hbm.at[pl.ds(base, bpw)], idx_v)
        wsems = (wsem0, wsem1)
        gathers = [None, None]
        writes = [None, None]
        gathers[0] = pltpu.async_copy(
            table_hbm.at[idx_v.at[pl.ds(0, ch)]], rows_v.at[0], gsem)
        for c in range(nch):
            b = c % 2
            nb = (c + 1) % 2
            gathers[b].wait()
            if c + 1 < nch:
                if writes[nb] is not None:
                    writes[nb].wait()
                gathers[nb] = pltpu.async_copy(
                    table_hbm.at[idx_v.at[pl.ds((c + 1) * ch, ch)]],
                    rows_v.at[nb], gsem)
            writes[b] = pltpu.async_copy(
                rows_v.at[b], out_hbm.at[pl.ds(base + c * ch, ch)], wsems[b])
        writes[(nch - 2) % 2].wait()
        writes[(nch - 1) % 2].wait()

    return k(table, idx)


def kernel(x, batch, w_init, b_init, w1m1, b1m1, w1m2, b1m2, w1g, b1g, w2m1,
           b2m1, w2m2, b2m2, w2g, b2g, wc1, bc1, wc2, bc2):
    n = x.shape[0]
    npad = ((n + 639) // 640) * 640
    pad = npad - n
    key_f = jnp.pad(x[:, 0:1], ((0, pad), (0, 0)))
    feats = jnp.pad(x[:, 1:], ((0, pad), (0, 0)))
    sel = (lax.broadcasted_iota(jnp.int32, (1, DA - H), 1) == 0
           ).astype(jnp.float32)

    def wsplit(wm, wg, bm, bg):
        wa = jnp.concatenate([wm[0:H], wg[0:H]], axis=1)
        wf = jnp.concatenate([wm[2 * H:2 * H + 1], wg[2 * H:2 * H + 1]],
                             axis=1)
        wb = jnp.concatenate([wm[H:2 * H], wg[H:2 * H]], axis=1)
        ba = jnp.concatenate([bm, bg])[None, :]
        return wa, wf, wb, ba

    wa1, wf1, wb1, ba1 = wsplit(w1m1, w1g, b1m1, b1g)
    wa2, wf2, wb2, ba2 = wsplit(w2m1, w2g, b2m1, b2g)

    ba_blk = 640
    full = lambda r, c: pl.BlockSpec((r, c), lambda i: (0, 0))
    h, sq, a1, hk1 = pl.pallas_call(
        _prep_kernel,
        grid=(npad // ba_blk,),
        in_specs=[
            pl.BlockSpec((ba_blk, H), lambda i: (i, 0)),
            pl.BlockSpec((ba_blk, 1), lambda i: (i, 0)),
            full(1, DA - H), full(H, H), full(1, H), full(H, 2 * H),
            full(1, 2 * H), full(1, 2 * H),
        ],
        out_specs=[
            pl.BlockSpec((ba_blk, H), lambda i: (i, 0)),
            pl.BlockSpec((ba_blk, 1), lambda i: (i, 0)),
            pl.BlockSpec((ba_blk, 2 * H), lambda i: (i, 0)),
            pl.BlockSpec((ba_blk, DA), lambda i: (i, 0)),
        ],
        out_shape=[
            jax.ShapeDtypeStruct((npad, H), jnp.float32),
            jax.ShapeDtypeStruct((npad, 1), jnp.float32),
            jax.ShapeDtypeStruct((npad, 2 * H), jnp.float32),
            jax.ShapeDtypeStruct((npad, DA), jnp.float32),
        ],
    )(feats, key_f, sel, w_init, b_init[None, :], wa1, wf1, ba1)

    br = 512
    idx = pl.pallas_call(
        functools.partial(_knn_kernel, n, br, npad),
        grid=(npad // br,),
        in_specs=[
            pl.BlockSpec((br, H), lambda i: (i, 0)),
            full(npad, H),
            pl.BlockSpec((br, 1), lambda i: (i, 0)),
            full(1, npad),
        ],
        out_specs=pl.BlockSpec((br, K), lambda i: (i, 0)),
        out_shape=jax.ShapeDtypeStruct((npad, K), jnp.int32),
        scratch_shapes=[pltpu.VMEM((br, npad), jnp.int32),
                        pltpu.VMEM((br, 1920), jnp.int32)],
    )(h, h, sq, sq.reshape(1, npad))

    src = idx.reshape(npad * K)
    bn = 256
    g1 = _sc_gather(hk1, src)
    x1, a2, hk2 = pl.pallas_call(
        functools.partial(_edge1_kernel, bn),
        grid=(npad // bn,),
        in_specs=[
            pl.BlockSpec((bn * K, DA), lambda i: (i, 0)),
            pl.BlockSpec((bn, 2 * H), lambda i: (i, 0)),
            pl.BlockSpec((bn, H), lambda i: (i, 0)),
            pl.BlockSpec((bn, 1), lambda i: (i, 0)),
            full(1, DA - H), full(H, 2 * H), full(1, 2 * H), full(H, H),
            full(1, H), full(H, 2 * H), full(1, 2 * H), full(1, 2 * H),
        ],
        out_specs=[
            pl.BlockSpec((bn, H), lambda i: (i, 0)),
            pl.BlockSpec((bn, 2 * H), lambda i: (i, 0)),
            pl.BlockSpec((bn, DA), lambda i: (i, 0)),
        ],
        out_shape=[
            jax.ShapeDtypeStruct((npad, H), jnp.float32),
            jax.ShapeDtypeStruct((npad, 2 * H), jnp.float32),
            jax.ShapeDtypeStruct((npad, DA), jnp.float32),
        ],
    )(g1, a1, h, key_f, sel, wb1, wf1, w1m2, b1m2[None, :], wa2, wf2, ba2)

    g2 = _sc_gather(hk2, src)
    pooled = pl.pallas_call(
        functools.partial(_edge2_kernel, n, bn),
        grid=(npad // bn,),
        in_specs=[
            pl.BlockSpec((bn * K, DA), lambda i: (i, 0)),
            pl.BlockSpec((bn, 2 * H), lambda i: (i, 0)),
            pl.BlockSpec((bn, H), lambda i: (i, 0)),
            full(H, 2 * H), full(1, 2 * H), full(H, H), full(1, H),
        ],
        out_specs=pl.BlockSpec((1, 2 * H), lambda i: (0, 0)),
        out_shape=jax.ShapeDtypeStruct((1, 2 * H), jnp.float32),
    )(g2, a2, x1, wb2, wf2, w2m2, b2m2[None, :])

    return pl.pallas_call(
        _cls_kernel,
        out_shape=jax.ShapeDtypeStruct((1, 2), jnp.float32),
    )(pooled, wc1, bc1[None, :], wc2, bc2[None, :])


# speedup vs baseline: 1.1573x; 1.1573x over previous
import functools

import jax
import jax.numpy as jnp
from jax import lax
from jax.experimental import pallas as pl
from jax.experimental.pallas import tpu as pltpu
from jax.experimental.pallas import tpu_sc as plsc

H = 128
K = 16
DA = 256
NEG = -3.0e38


def _prep_kernel(feats_ref, key_ref, sel_ref, wi_ref, bi_ref, wa_ref, wf_ref,
                 ba_ref, h_ref, sq_ref, a_ref, hk_ref):
    f = feats_ref[...]
    h = jnp.maximum(f @ wi_ref[...] + bi_ref[...], 0.0)
    kv = key_ref[...]
    h_ref[...] = h
    sq_ref[...] = jnp.sum(h * h, axis=1, keepdims=True)
    a_ref[...] = h @ wa_ref[...] + kv * wf_ref[...] + ba_ref[...]
    hk_ref[:, 0:H] = h
    hk_ref[:, H:DA] = kv * sel_ref[...]


def _knn_kernel(nreal, br, npad, hb_ref, ha_ref, sqb_ref, sqt_ref, idx_ref,
                k_ref, c_ref):
    i = pl.program_id(0)
    maxi = jnp.int32(0x7FFFFFFF)
    ngrp = npad // 640
    hb = hb_ref[...]
    d = sqb_ref[...] + sqt_ref[...] + lax.dot_general(
        -2.0 * hb, ha_ref[...], (((1,), (1,)), ((), ())),
        preferred_element_type=jnp.float32)
    rows = i * br + lax.broadcasted_iota(jnp.int32, (br, npad), 0)
    cols = lax.broadcasted_iota(jnp.int32, (br, npad), 1)
    rm = jnp.max(d, axis=1, keepdims=True)
    q = (jnp.maximum(d, 0.0) * (131071.0 / jnp.maximum(rm, 1e-20))
         ).astype(jnp.int32)
    key = (q << 14) | cols
    key = jnp.where((cols == rows) | (cols >= nreal), maxi, key)
    k_ref[...] = key
    sl = [key[:, j * 640:(j + 1) * 640] for j in range(ngrp)]
    m1 = sl[0]
    for j in range(1, ngrp):
        m1 = jnp.minimum(m1, sl[j])
    m2 = maxi
    for j in range(ngrp):
        m2 = jnp.minimum(m2, jnp.where(sl[j] == m1, maxi, sl[j]))
    m3 = maxi
    for j in range(ngrp):
        m3 = jnp.minimum(
            m3, jnp.where((sl[j] == m1) | (sl[j] == m2), maxi, sl[j]))
    c_ref[:, 0:640] = m1
    c_ref[:, 640:1280] = m2
    c_ref[:, 1280:1920] = m3
    prev = jnp.full((br, 1), -1, jnp.int32)
    for t in range(K):
        kv = c_ref[...]
        m = jnp.min(jnp.where(kv > prev, kv, maxi), axis=1, keepdims=True)
        idx_ref[:, t:t + 1] = m & 0x3FFF
        prev = m
    cnt = jnp.zeros((br, 640), jnp.int32)
    for j in range(ngrp):
        cnt = cnt + jnp.where(sl[j] <= prev, 1, 0)
    bad = jnp.any(jnp.sum(cnt, axis=1, keepdims=True) != K)

    @pl.when(bad)
    def _():
        prev = jnp.full((br, 1), -1, jnp.int32)
        for t in range(K):
            kv = k_ref[...]
            m = jnp.min(jnp.where(kv > prev, kv, maxi), axis=1, keepdims=True)
            idx_ref[:, t:t + 1] = m & 0x3FFF
            prev = m


def _edge1_kernel(bn, g_ref, a_ref, h_ref, key_ref, sel_ref, wb_ref, wf_ref,
                  wm2_ref, bm2_ref, wa2_ref, wf2_ref, ba2_ref, x1_ref, a2_ref,
                  hk2_ref):
    g = g_ref[...]
    a = a_ref[...]
    a3 = jnp.broadcast_to(a[:, None, :], (bn, K, 2 * H)).reshape(bn * K, 2 * H)
    pre = g[:, 0:H] @ wb_ref[...] + a3 - g[:, H:H + 1] * wf_ref[...]
    tmsg = jnp.maximum(pre[:, 0:H], 0.0) @ wm2_ref[...] + bm2_ref[...]
    gate = 1.0 / (1.0 + jnp.exp(-pre[:, H:2 * H]))
    msg = gate * tmsg + (1.0 - gate) * g[:, 0:H]
    s = jnp.sum(msg.reshape(bn, K, H), axis=1) * (1.0 / K)
    x = s + h_ref[...]
    x = jnp.where(x > 0, x, 0.2 * x)
    kv = key_ref[...]
    x1_ref[...] = x
    a2_ref[...] = x @ wa2_ref[...] + kv * wf2_ref[...] + ba2_ref[...]
    hk2_ref[:, 0:H] = x
    hk2_ref[:, H:DA] = kv * sel_ref[...]


def _edge2_kernel(nreal, bn, g_ref, a_ref, x1_ref, wb_ref, wf_ref, wm2_ref,
                  bm2_ref, pool_ref):
    i = pl.program_id(0)
    g = g_ref[...]
    a = a_ref[...]
    a3 = jnp.broadcast_to(a[:, None, :], (bn, K, 2 * H)).reshape(bn * K, 2 * H)
    pre = g[:, 0:H] @ wb_ref[...] + a3 - g[:, H:H + 1] * wf_ref[...]
    tmsg = jnp.maximum(pre[:, 0:H], 0.0) @ wm2_ref[...] + bm2_ref[...]
    gate = 1.0 / (1.0 + jnp.exp(-pre[:, H:2 * H]))
    msg = gate * tmsg + (1.0 - gate) * g[:, 0:H]
    s = jnp.sum(msg.reshape(bn, K, H), axis=1) * (1.0 / K)
    x1 = x1_ref[...]
    x2 = s + x1
    x2 = jnp.where(x2 > 0, x2, 0.2 * x2)
    xc = jnp.concatenate([x1, x2], axis=1)
    rows = i * bn + lax.broadcasted_iota(jnp.int32, (bn, 1), 0)
    xm = jnp.where(rows < nreal, xc, NEG)
    bmax = jnp.max(xm, axis=0, keepdims=True)

    @pl.when(i == 0)
    def _():
        pool_ref[...] = bmax

    @pl.when(i > 0)
    def _():
        pool_ref[...] = jnp.maximum(pool_ref[...], bmax)


def _cls_kernel(p_ref, w1_ref, b1_ref, w2_ref, b2_ref, o_ref):
    z = jnp.maximum(p_ref[...] @ w1_ref[...] + b1_ref[...], 0.0)
    z = z @ w2_ref[...] + b2_ref[...]
    m = jnp.max(z, axis=1, keepdims=True)
    e = jnp.exp(z - m)
    o_ref[...] = (z - m) - jnp.log(jnp.sum(e, axis=1, keepdims=True))


def _sc_gather(table, idx):
    ne = idx.shape[0]
    d = table.shape[1]
    nw = 32
    bpw = ne // nw
    ch = 160
    nch = bpw // ch
    mesh = plsc.VectorSubcoreMesh(core_axis_name="c", subcore_axis_name="s")

    @functools.partial(
        pl.kernel,
        out_type=jax.ShapeDtypeStruct((ne, d), jnp.float32),
        mesh=mesh,
        scratch_types=[
            pltpu.VMEM((bpw,), jnp.int32),
            pltpu.VMEM((2, ch, d), jnp.float32),
            pltpu.SemaphoreType.DMA,
            pltpu.SemaphoreType.DMA,
            pltpu.SemaphoreType.DMA,
        ],
    )
    def k(table_hbm, idx_hbm, out_hbm, idx_v, rows_v, gsem, wsem0, wsem1):
        wid = lax.axis_index("s") * 2 + lax.axis_index("c")
        base = wid * bpw
        pltpu.sync_copy(idx_hbm.at[pl.ds(base, bpw)], idx_v)
        wsems = (wsem0, wsem1)
        gathers = [None, None]
        writes = [None, None]
        gathers[0] = pltpu.async_copy(
            table_hbm.at[idx_v.at[pl.ds(0, ch)]], rows_v.at[0], gsem)
        for c in range(nch):
            b = c % 2
            nb = (c + 1) % 2
            gathers[b].wait()
            if c + 1 < nch:
                if writes[nb] is not None:
                    writes[nb].wait()
                gathers[nb] = pltpu.async_copy(
                    table_hbm.at[idx_v.at[pl.ds((c + 1) * ch, ch)]],
                    rows_v.at[nb], gsem)
            writes[b] = pltpu.async_copy(
                rows_v.at[b], out_hbm.at[pl.ds(base + c * ch, ch)], wsems[b])
        writes[(nch - 2) % 2].wait()
        writes[(nch - 1) % 2].wait()

    return k(table, idx)


def kernel(x, batch, w_init, b_init, w1m1, b1m1, w1m2, b1m2, w1g, b1g, w2m1,
           b2m1, w2m2, b2m2, w2g, b2g, wc1, bc1, wc2, bc2):
    n = x.shape[0]
    npad = ((n + 639) // 640) * 640
    pad = npad - n
    key_f = jnp.pad(x[:, 0:1], ((0, pad), (0, 0)))
    feats = jnp.pad(x[:, 1:], ((0, pad), (0, 0)))
    sel = (lax.broadcasted_iota(jnp.int32, (1, DA - H), 1) == 0
           ).astype(jnp.float32)

    def wsplit(wm, wg, bm, bg):
        wa = jnp.concatenate([wm[0:H], wg[0:H]], axis=1)
        wf = jnp.concatenate([wm[2 * H:2 * H + 1], wg[2 * H:2 * H + 1]],
                             axis=1)
        wb = jnp.concatenate([wm[H:2 * H], wg[H:2 * H]], axis=1)
        ba = jnp.concatenate([bm, bg])[None, :]
        return wa, wf, wb, ba

    wa1, wf1, wb1, ba1 = wsplit(w1m1, w1g, b1m1, b1g)
    wa2, wf2, wb2, ba2 = wsplit(w2m1, w2g, b2m1, b2g)

    ba_blk = 640
    full = lambda r, c: pl.BlockSpec((r, c), lambda i: (0, 0))
    h, sq, a1, hk1 = pl.pallas_call(
        _prep_kernel,
        grid=(npad // ba_blk,),
        in_specs=[
            pl.BlockSpec((ba_blk, H), lambda i: (i, 0)),
            pl.BlockSpec((ba_blk, 1), lambda i: (i, 0)),
            full(1, DA - H), full(H, H), full(1, H), full(H, 2 * H),
            full(1, 2 * H), full(1, 2 * H),
        ],
        out_specs=[
            pl.BlockSpec((ba_blk, H), lambda i: (i, 0)),
            pl.BlockSpec((ba_blk, 1), lambda i: (i, 0)),
            pl.BlockSpec((ba_blk, 2 * H), lambda i: (i, 0)),
            pl.BlockSpec((ba_blk, DA), lambda i: (i, 0)),
        ],
        out_shape=[
            jax.ShapeDtypeStruct((npad, H), jnp.float32),
            jax.ShapeDtypeStruct((npad, 1), jnp.float32),
            jax.ShapeDtypeStruct((npad, 2 * H), jnp.float32),
            jax.ShapeDtypeStruct((npad, DA), jnp.float32),
        ],
    )(feats, key_f, sel, w_init, b_init[None, :], wa1, wf1, ba1)

    br = 256
    idx = pl.pallas_call(
        functools.partial(_knn_kernel, n, br, npad),
        grid=(npad // br,),
        in_specs=[
            pl.BlockSpec((br, H), lambda i: (i, 0)),
            full(npad, H),
            pl.BlockSpec((br, 1), lambda i: (i, 0)),
            full(1, npad),
        ],
        out_specs=pl.BlockSpec((br, K), lambda i: (i, 0)),
        out_shape=jax.ShapeDtypeStruct((npad, K), jnp.int32),
        scratch_shapes=[pltpu.VMEM((br, npad), jnp.int32),
                        pltpu.VMEM((br, 1920), jnp.int32)],
    )(h, h, sq, sq.reshape(1, npad))

    src = idx.reshape(npad * K)
    bn = 256
    g1 = _sc_gather(hk1, src)
    x1, a2, hk2 = pl.pallas_call(
        functools.partial(_edge1_kernel, bn),
        grid=(npad // bn,),
        in_specs=[
            pl.BlockSpec((bn * K, DA), lambda i: (i, 0)),
            pl.BlockSpec((bn, 2 * H), lambda i: (i, 0)),
            pl.BlockSpec((bn, H), lambda i: (i, 0)),
            pl.BlockSpec((bn, 1), lambda i: (i, 0)),
            full(1, DA - H), full(H, 2 * H), full(1, 2 * H), full(H, H),
            full(1, H), full(H, 2 * H), full(1, 2 * H), full(1, 2 * H),
        ],
        out_specs=[
            pl.BlockSpec((bn, H), lambda i: (i, 0)),
            pl.BlockSpec((bn, 2 * H), lambda i: (i, 0)),
            pl.BlockSpec((bn, DA), lambda i: (i, 0)),
        ],
        out_shape=[
            jax.ShapeDtypeStruct((npad, H), jnp.float32),
            jax.ShapeDtypeStruct((npad, 2 * H), jnp.float32),
            jax.ShapeDtypeStruct((npad, DA), jnp.float32),
        ],
    )(g1, a1, h, key_f, sel, wb1, wf1, w1m2, b1m2[None, :], wa2, wf2, ba2)

    g2 = _sc_gather(hk2, src)
    pooled = pl.pallas_call(
        functools.partial(_edge2_kernel, n, bn),
        grid=(npad // bn,),
        in_specs=[
            pl.BlockSpec((bn * K, DA), lambda i: (i, 0)),
            pl.BlockSpec((bn, 2 * H), lambda i: (i, 0)),
            pl.BlockSpec((bn, H), lambda i: (i, 0)),
            full(H, 2 * H), full(1, 2 * H), full(H, H), full(1, H),
        ],
        out_specs=pl.BlockSpec((1, 2 * H), lambda i: (0, 0)),
        out_shape=jax.ShapeDtypeStruct((1, 2 * H), jnp.float32),
    )(g2, a2, x1, wb2, wf2, w2m2, b2m2[None, :])

    return pl.pallas_call(
        _cls_kernel,
        out_shape=jax.ShapeDtypeStruct((1, 2), jnp.float32),
    )(pooled, wc1, bc1[None, :], wc2, bc2[None, :])
